# baseline (speedup 1.0000x reference)
import functools

import jax
import jax.numpy as jnp
from jax import lax
from jax.experimental import pallas as pl
from jax.experimental.pallas import tpu as pltpu
from jax.experimental.pallas import tpu_sc as plsc

E = 64
C = 8
K = 16
L = 16
NC = 2
NS = 16
NW = NC * NS
UNROLL = 4


def _build_sc_kernel(B: int):
  R = B // NW
  mesh = plsc.VectorSubcoreMesh(core_axis_name="c", subcore_axis_name="s")

  @functools.partial(
      pl.kernel,
      out_type=(
          jax.ShapeDtypeStruct((B * K,), jnp.int32),
          jax.ShapeDtypeStruct((B * K,), jnp.float32),
      ),
      mesh=mesh,
      compiler_params=pltpu.CompilerParams(needs_layout_passes=False),
      scratch_types=[
          pltpu.VMEM((E * E,), jnp.float32),
          pltpu.VMEM((R * C + L,), jnp.int32),
          pltpu.VMEM((R * C + L,), jnp.float32),
          pltpu.VMEM((L,), jnp.int32),
          pltpu.VMEM((L,), jnp.float32),
          pltpu.VMEM((R * K,), jnp.int32),
          pltpu.VMEM((R * K,), jnp.float32),
      ],
  )
  def sc_kernel(scores_hbm, cooc_hbm, ids_hbm, dvi_hbm, dvf_hbm,
                oi_hbm, os_hbm,
                cooc_v, ids_v, sc_v, di_v, df_v, oi_v, os_v):
    wid = lax.axis_index("s") * NC + lax.axis_index("c")
    base = wid * R
    pltpu.sync_copy(cooc_hbm, cooc_v)
    pltpu.sync_copy(ids_hbm.at[pl.ds(base * C, R * C)],
                    ids_v.at[pl.ds(0, R * C)])
    pltpu.sync_copy(scores_hbm.at[pl.ds(base * C, R * C)],
                    sc_v.at[pl.ds(0, R * C)])
    pltpu.sync_copy(dvi_hbm, di_v)
    pltpu.sync_copy(dvf_hbm, df_v)

    lane = lax.iota(jnp.int32, L)
    mask_lo = lane < C
    neg_inf = jnp.full((L,), -jnp.inf, jnp.float32)
    vals = [lane + j * L for j in range(E // L)]
    di = di_v[...]
    df = df_v[...]
    ids_v[pl.ds(R * C, L)] = jnp.zeros((L,), jnp.int32)

    def one_row(r):
      o8 = r * C
      ids16 = ids_v[pl.ds(o8, L)]
      s16 = sc_v[pl.ds(o8, L)]
      cids = [ids16[i] for i in range(C)]
      acc_a = None
      acc_b = None
      for i in range(C // 2):
        sa, sb = s16[i], s16[i + C // 2]
        cba = cids[i] * E
        cbb = cids[i + C // 2] * E
        rows_a = [cooc_v[pl.ds(cba + j * L, L)] for j in range(E // L)]
        rows_b = [cooc_v[pl.ds(cbb + j * L, L)] for j in range(E // L)]
        if acc_a is None:
          acc_a = [sa * rj for rj in rows_a]
          acc_b = [sb * rj for rj in rows_b]
        else:
          acc_a = [a + sa * rj for a, rj in zip(acc_a, rows_a)]
          acc_b = [a + sb * rj for a, rj in zip(acc_b, rows_b)]
      accs = [a + b for a, b in zip(acc_a, acc_b)]
      for i in range(C):
        accs = [jnp.where(vj == cids[i], neg_inf, aj)
                for vj, aj in zip(vals, accs)]
      s0k, s0v = plsc.sort_key_val(accs[0], vals[0], descending=True)
      s1k, s1v = plsc.sort_key_val(accs[1], vals[1], descending=False)
      s2k, s2v = plsc.sort_key_val(accs[2], vals[2], descending=True)
      s3k, s3v = plsc.sort_key_val(accs[3], vals[3], descending=False)
      t01k, t01v = plsc.sort_key_val(jnp.where(mask_lo, s0k, s1k),
                                     jnp.where(mask_lo, s0v, s1v),
                                     descending=True)
      t23k, t23v = plsc.sort_key_val(jnp.where(mask_lo, s2k, s3k),
                                     jnp.where(mask_lo, s2v, s3v),
                                     descending=False)
      fk, fv = plsc.sort_key_val(jnp.where(mask_lo, t01k, t23k),
                                 jnp.where(mask_lo, t01v, t23v),
                                 descending=True)
      rk = r * K
      oi_v[pl.ds(rk, L)] = ids16 + di
      os_v[pl.ds(rk, L)] = s16 + df
      hi_idx = lane + (rk + C)
      plsc.store_scatter(oi_v, [hi_idx], fv + di, mask=mask_lo)
      plsc.store_scatter(os_v, [hi_idx], fk + df, mask=mask_lo)

    def row_body(r, carry):
      one_row(2 * r)
      one_row(2 * r + 1)
      return carry

    lax.fori_loop(0, R // 2, row_body, 0)
    pltpu.sync_copy(oi_v, oi_hbm.at[pl.ds(base * K, R * K)])
    pltpu.sync_copy(os_v, os_hbm.at[pl.ds(base * K, R * K)])

  return sc_kernel


@functools.cache
def _get_sc_kernel(B: int):
  return _build_sc_kernel(B)


def kernel(candidate_scores, cooccurrence, candidate_ids, target_size):
  B, _ = candidate_ids.shape
  delta_i = jnp.asarray(target_size, jnp.int32) - K
  dvi = jnp.full((L,), delta_i, jnp.int32)
  dvf = jnp.full((L,), delta_i.astype(jnp.float32), jnp.float32)
  oi, os_ = _get_sc_kernel(B)(
      candidate_scores.reshape(-1),
      cooccurrence.reshape(-1),
      candidate_ids.reshape(-1),
      dvi,
      dvf,
  )
  return oi.reshape(B, K), os_.reshape(B, K)


# speedup vs baseline: 1.0148x; 1.0148x over previous
import functools

import jax
import jax.numpy as jnp
from jax import lax
from jax.experimental import pallas as pl
from jax.experimental.pallas import tpu as pltpu
from jax.experimental.pallas import tpu_sc as plsc

E = 64
C = 8
K = 16
L = 16
NC = 2
NS = 16
NW = NC * NS
UNROLL = 4


def _build_sc_kernel(B: int):
  R = B // NW
  mesh = plsc.VectorSubcoreMesh(core_axis_name="c", subcore_axis_name="s")

  @functools.partial(
      pl.kernel,
      out_type=(
          jax.ShapeDtypeStruct((B * K,), jnp.int32),
          jax.ShapeDtypeStruct((B * K,), jnp.float32),
      ),
      mesh=mesh,
      compiler_params=pltpu.CompilerParams(needs_layout_passes=False),
      scratch_types=[
          pltpu.VMEM((E * E,), jnp.float32),
          pltpu.VMEM((R * C + L,), jnp.int32),
          pltpu.VMEM((R * C + L,), jnp.float32),
          pltpu.VMEM((L,), jnp.int32),
          pltpu.VMEM((L,), jnp.float32),
          pltpu.VMEM((R * K,), jnp.int32),
          pltpu.VMEM((R * K,), jnp.float32),
      ],
  )
  def sc_kernel(scores_hbm, cooc_hbm, ids_hbm, dvi_hbm, dvf_hbm,
                oi_hbm, os_hbm,
                cooc_v, ids_v, sc_v, di_v, df_v, oi_v, os_v):
    wid = lax.axis_index("s") * NC + lax.axis_index("c")
    base = wid * R
    pltpu.sync_copy(cooc_hbm, cooc_v)
    pltpu.sync_copy(ids_hbm.at[pl.ds(base * C, R * C)],
                    ids_v.at[pl.ds(0, R * C)])
    pltpu.sync_copy(scores_hbm.at[pl.ds(base * C, R * C)],
                    sc_v.at[pl.ds(0, R * C)])
    pltpu.sync_copy(dvi_hbm, di_v)
    pltpu.sync_copy(dvf_hbm, df_v)

    lane = lax.iota(jnp.int32, L)
    mask_lo = lane < C
    neg_inf = jnp.full((L,), -jnp.inf, jnp.float32)
    vals = [lane + j * L for j in range(E // L)]
    di = di_v[...]
    df = df_v[...]
    ids_v[pl.ds(R * C, L)] = jnp.zeros((L,), jnp.int32)

    def one_row(r):
      o8 = r * C
      ids16 = ids_v[pl.ds(o8, L)]
      s16 = sc_v[pl.ds(o8, L)]
      idvecs = [plsc.load_gather(ids_v, [jnp.full((L,), o8 + i, jnp.int32)])
                for i in range(C)]
      svecs = [plsc.load_gather(sc_v, [jnp.full((L,), o8 + i, jnp.int32)])
               for i in range(C)]
      acc_a = None
      acc_b = None
      for i in range(C // 2):
        ia, ib = i, i + C // 2
        cba = idvecs[ia] * E
        cbb = idvecs[ib] * E
        rows_a = [plsc.load_gather(cooc_v, [cba + vj]) for vj in vals]
        rows_b = [plsc.load_gather(cooc_v, [cbb + vj]) for vj in vals]
        if acc_a is None:
          acc_a = [svecs[ia] * rj for rj in rows_a]
          acc_b = [svecs[ib] * rj for rj in rows_b]
        else:
          acc_a = [a + svecs[ia] * rj for a, rj in zip(acc_a, rows_a)]
          acc_b = [a + svecs[ib] * rj for a, rj in zip(acc_b, rows_b)]
      accs = [a + b for a, b in zip(acc_a, acc_b)]
      for i in range(C):
        accs = [jnp.where(vj == idvecs[i], neg_inf, aj)
                for vj, aj in zip(vals, accs)]
      s0k, s0v = plsc.sort_key_val(accs[0], vals[0], descending=True)
      s1k, s1v = plsc.sort_key_val(accs[1], vals[1], descending=False)
      s2k, s2v = plsc.sort_key_val(accs[2], vals[2], descending=True)
      s3k, s3v = plsc.sort_key_val(accs[3], vals[3], descending=False)
      t01k, t01v = plsc.sort_key_val(jnp.where(mask_lo, s0k, s1k),
                                     jnp.where(mask_lo, s0v, s1v),
                                     descending=True)
      t23k, t23v = plsc.sort_key_val(jnp.where(mask_lo, s2k, s3k),
                                     jnp.where(mask_lo, s2v, s3v),
                                     descending=False)
      fk, fv = plsc.sort_key_val(jnp.where(mask_lo, t01k, t23k),
                                 jnp.where(mask_lo, t01v, t23v),
                                 descending=True)
      rk = r * K
      oi_v[pl.ds(rk, L)] = ids16 + di
      os_v[pl.ds(rk, L)] = s16 + df
      hi_idx = lane + (rk + C)
      plsc.store_scatter(oi_v, [hi_idx], fv + di, mask=mask_lo)
      plsc.store_scatter(os_v, [hi_idx], fk + df, mask=mask_lo)

    def row_body(r, carry):
      one_row(r)
      return carry

    lax.fori_loop(0, R, row_body, 0)
    pltpu.sync_copy(oi_v, oi_hbm.at[pl.ds(base * K, R * K)])
    pltpu.sync_copy(os_v, os_hbm.at[pl.ds(base * K, R * K)])

  return sc_kernel


@functools.cache
def _get_sc_kernel(B: int):
  return _build_sc_kernel(B)


def kernel(candidate_scores, cooccurrence, candidate_ids, target_size):
  B, _ = candidate_ids.shape
  delta_i = jnp.asarray(target_size, jnp.int32) - K
  dvi = jnp.full((L,), delta_i, jnp.int32)
  dvf = jnp.full((L,), delta_i.astype(jnp.float32), jnp.float32)
  oi, os_ = _get_sc_kernel(B)(
      candidate_scores.reshape(-1),
      cooccurrence.reshape(-1),
      candidate_ids.reshape(-1),
      dvi,
      dvf,
  )
  return oi.reshape(B, K), os_.reshape(B, K)


# speedup vs baseline: 1.0209x; 1.0059x over previous
import functools

import jax
import jax.numpy as jnp
from jax import lax
from jax.experimental import pallas as pl
from jax.experimental.pallas import tpu as pltpu
from jax.experimental.pallas import tpu_sc as plsc

E = 64
C = 8
K = 16
L = 16
NC = 2
NS = 16
NW = NC * NS


def _build_sc_kernel(B: int):
  R = B // NW
  mesh = plsc.VectorSubcoreMesh(core_axis_name="c", subcore_axis_name="s")

  @functools.partial(
      pl.kernel,
      out_type=(
          jax.ShapeDtypeStruct((B, K), jnp.int32),
          jax.ShapeDtypeStruct((B, K), jnp.float32),
      ),
      mesh=mesh,
      compiler_params=pltpu.CompilerParams(
          needs_layout_passes=False, use_tc_tiling_on_sc=False),
      scratch_types=[
          pltpu.VMEM((E, E), jnp.float32),
          pltpu.VMEM((R, C), jnp.int32),
          pltpu.VMEM((R, C), jnp.float32),
          pltpu.VMEM((L,), jnp.int32),
          pltpu.VMEM((L,), jnp.float32),
          pltpu.VMEM((R, K), jnp.int32),
          pltpu.VMEM((R, K), jnp.float32),
      ],
  )
  def sc_kernel(scores_hbm, cooc_hbm, ids_hbm, dvi_hbm, dvf_hbm,
                oi_hbm, os_hbm,
                cooc_v, ids_v, sc_v, di_v, df_v, oi_v, os_v):
    wid = lax.axis_index("s") * NC + lax.axis_index("c")
    base = wid * R
    pltpu.sync_copy(cooc_hbm, cooc_v)
    pltpu.sync_copy(ids_hbm.at[pl.ds(base, R)], ids_v)
    pltpu.sync_copy(scores_hbm.at[pl.ds(base, R)], sc_v)
    pltpu.sync_copy(dvi_hbm, di_v)
    pltpu.sync_copy(dvf_hbm, df_v)

    lane = lax.iota(jnp.int32, L)
    mask_lo = lane < C
    neg_inf = jnp.full((L,), -jnp.inf, jnp.float32)
    vals = [lane + j * L for j in range(E // L)]
    col8 = lane & (C - 1)
    ocol = col8 + C
    splats = [jnp.full((L,), i, jnp.int32) for i in range(C)]
    di = di_v[...]
    df = df_v[...]

    def one_row(r):
      rvec = jnp.full((L,), 0, jnp.int32) + r
      ids16 = plsc.load_gather(ids_v, [rvec, col8])
      s16 = plsc.load_gather(sc_v, [rvec, col8])
      idvecs = [plsc.load_gather(ids_v, [rvec, splats[i]]) for i in range(C)]
      svecs = [plsc.load_gather(sc_v, [rvec, splats[i]]) for i in range(C)]
      acc_a = None
      acc_b = None
      for i in range(C // 2):
        ia, ib = i, i + C // 2
        rows_a = [plsc.load_gather(cooc_v, [idvecs[ia], vj]) for vj in vals]
        rows_b = [plsc.load_gather(cooc_v, [idvecs[ib], vj]) for vj in vals]
        if acc_a is None:
          acc_a = [svecs[ia] * rj for rj in rows_a]
          acc_b = [svecs[ib] * rj for rj in rows_b]
        else:
          acc_a = [a + svecs[ia] * rj for a, rj in zip(acc_a, rows_a)]
          acc_b = [a + svecs[ib] * rj for a, rj in zip(acc_b, rows_b)]
      accs = [a + b for a, b in zip(acc_a, acc_b)]
      for i in range(C):
        accs = [jnp.where(vj == idvecs[i], neg_inf, aj)
                for vj, aj in zip(vals, accs)]
      s0k, s0v = plsc.sort_key_val(accs[0], vals[0], descending=True)
      s1k, s1v = plsc.sort_key_val(accs[1], vals[1], descending=False)
      s2k, s2v = plsc.sort_key_val(accs[2], vals[2], descending=True)
      s3k, s3v = plsc.sort_key_val(accs[3], vals[3], descending=False)
      t01k, t01v = plsc.sort_key_val(jnp.where(mask_lo, s0k, s1k),
                                     jnp.where(mask_lo, s0v, s1v),
                                     descending=True)
      t23k, t23v = plsc.sort_key_val(jnp.where(mask_lo, s2k, s3k),
                                     jnp.where(mask_lo, s2v, s3v),
                                     descending=False)
      fk, fv = plsc.sort_key_val(jnp.where(mask_lo, t01k, t23k),
                                 jnp.where(mask_lo, t01v, t23v),
                                 descending=True)
      oi_v[r] = ids16 + di
      os_v[r] = s16 + df
      plsc.store_scatter(oi_v, [rvec, ocol], fv + di, mask=mask_lo)
      plsc.store_scatter(os_v, [rvec, ocol], fk + df, mask=mask_lo)

    def row_body(r, carry):
      one_row(r)
      return carry

    lax.fori_loop(0, R, row_body, 0)
    pltpu.sync_copy(oi_v, oi_hbm.at[pl.ds(base, R)])
    pltpu.sync_copy(os_v, os_hbm.at[pl.ds(base, R)])

  return sc_kernel


@functools.cache
def _get_sc_kernel(B: int):
  return _build_sc_kernel(B)


def kernel(candidate_scores, cooccurrence, candidate_ids, target_size):
  B, _ = candidate_ids.shape
  delta_i = jnp.asarray(target_size, jnp.int32) - K
  dvi = jnp.full((L,), delta_i, jnp.int32)
  dvf = jnp.full((L,), delta_i.astype(jnp.float32), jnp.float32)
  return _get_sc_kernel(B)(
      candidate_scores,
      cooccurrence,
      candidate_ids,
      dvi,
      dvf,
  )


# speedup vs baseline: 1.1994x; 1.1749x over previous
import functools

import jax
import jax.numpy as jnp
from jax import lax
from jax.experimental import pallas as pl
from jax.experimental.pallas import tpu as pltpu
from jax.experimental.pallas import tpu_sc as plsc

E = 64
C = 8
K = 16
L = 16
NC = 2
NS = 16
NW = NC * NS


def _build_sc_kernel(B: int):
  R = B // NW
  mesh = plsc.VectorSubcoreMesh(core_axis_name="c", subcore_axis_name="s")

  @functools.partial(
      pl.kernel,
      out_type=(
          jax.ShapeDtypeStruct((B, K), jnp.int32),
          jax.ShapeDtypeStruct((B, K), jnp.float32),
      ),
      mesh=mesh,
      compiler_params=pltpu.CompilerParams(
          needs_layout_passes=False, use_tc_tiling_on_sc=False),
      scratch_types=[
          pltpu.VMEM((E, E), jnp.float32),
          pltpu.VMEM((R, C), jnp.int32),
          pltpu.VMEM((R, C), jnp.float32),
          pltpu.VMEM((L,), jnp.int32),
          pltpu.VMEM((L,), jnp.float32),
          pltpu.VMEM((R, K), jnp.int32),
          pltpu.VMEM((R, K), jnp.float32),
          pltpu.SemaphoreType.DMA,
      ],
  )
  def sc_kernel(scores_hbm, cooc_hbm, ids_hbm, dvi_hbm, dvf_hbm,
                oi_hbm, os_hbm,
                cooc_v, ids_v, sc_v, di_v, df_v, oi_v, os_v, dsem):
    wid = lax.axis_index("s") * NC + lax.axis_index("c")
    base = wid * R
    copies = [
        pltpu.make_async_copy(cooc_hbm, cooc_v, dsem),
        pltpu.make_async_copy(ids_hbm.at[pl.ds(base, R)], ids_v, dsem),
        pltpu.make_async_copy(scores_hbm.at[pl.ds(base, R)], sc_v, dsem),
        pltpu.make_async_copy(dvi_hbm, di_v, dsem),
        pltpu.make_async_copy(dvf_hbm, df_v, dsem),
    ]
    for cp in copies:
      cp.start()
    for cp in copies:
      cp.wait()

    lane = lax.iota(jnp.int32, L)
    mask_lo = lane < C
    neg_inf = jnp.full((L,), -jnp.inf, jnp.float32)
    vals = [lane + j * L for j in range(E // L)]
    col8 = lane & (C - 1)
    ocol = col8 + C
    splats = [jnp.full((L,), i, jnp.int32) for i in range(C)]
    di = di_v[...]
    df = df_v[...]

    def accum(r):
      rvec = jnp.full((L,), 0, jnp.int32) + r
      prods = [[] for _ in range(E // L)]
      masks = [[] for _ in range(E // L)]
      for i in range(C):
        idv = plsc.load_gather(ids_v, [rvec, splats[i]])
        sv = plsc.load_gather(sc_v, [rvec, splats[i]])
        for j in range(E // L):
          rowj = plsc.load_gather(cooc_v, [idv, vals[j]])
          prods[j].append(sv * rowj)
          masks[j].append(vals[j] == idv)

      def tree(xs, op):
        while len(xs) > 1:
          xs = [op(xs[k], xs[k + 1]) for k in range(0, len(xs) - 1, 2)] + (
              [xs[-1]] if len(xs) & 1 else [])
        return xs[0]

      accs = tuple(
          jnp.where(tree(masks[j], jnp.logical_or), neg_inf,
                    tree(prods[j], jnp.add))
          for j in range(E // L))
      ids16 = plsc.load_gather(ids_v, [rvec, col8])
      s16 = plsc.load_gather(sc_v, [rvec, col8])
      return accs + (ids16, s16)

    def level1(state):
      a0, a1, a2, a3, ids16, s16 = state
      s0 = plsc.sort_key_val(a0, vals[0], descending=True)
      s1 = plsc.sort_key_val(a1, vals[1], descending=False)
      s2 = plsc.sort_key_val(a2, vals[2], descending=True)
      s3 = plsc.sort_key_val(a3, vals[3], descending=False)
      return s0, s1, s2, s3, ids16, s16

    def finish(r, state):
      (s0k, s0v), (s1k, s1v), (s2k, s2v), (s3k, s3v), ids16, s16 = state
      t01k, t01v = plsc.sort_key_val(jnp.where(mask_lo, s0k, s1k),
                                     jnp.where(mask_lo, s0v, s1v),
                                     descending=True)
      t23k, t23v = plsc.sort_key_val(jnp.where(mask_lo, s2k, s3k),
                                     jnp.where(mask_lo, s2v, s3v),
                                     descending=False)
      fk, fv = plsc.sort_key_val(jnp.where(mask_lo, t01k, t23k),
                                 jnp.where(mask_lo, t01v, t23v),
                                 descending=True)
      rvec = jnp.full((L,), 0, jnp.int32) + r
      oi_v[r] = ids16 + di
      os_v[r] = s16 + df
      plsc.store_scatter(oi_v, [rvec, ocol], fv + di, mask=mask_lo)
      plsc.store_scatter(os_v, [rvec, ocol], fk + df, mask=mask_lo)

    def row_body(r, carry):
      sorted1 = level1(carry)
      nxt = accum(r)
      finish(r - 1, sorted1)
      return nxt

    last = lax.fori_loop(1, R, row_body, accum(0))
    finish(R - 1, level1(last))
    pltpu.sync_copy(oi_v, oi_hbm.at[pl.ds(base, R)])
    pltpu.sync_copy(os_v, os_hbm.at[pl.ds(base, R)])

  return sc_kernel


@functools.cache
def _get_sc_kernel(B: int):
  return _build_sc_kernel(B)


def kernel(candidate_scores, cooccurrence, candidate_ids, target_size):
  B, _ = candidate_ids.shape
  delta_i = jnp.asarray(target_size, jnp.int32) - K
  dvi = jnp.full((L,), delta_i, jnp.int32)
  dvf = jnp.full((L,), delta_i.astype(jnp.float32), jnp.float32)
  return _get_sc_kernel(B)(
      candidate_scores,
      cooccurrence,
      candidate_ids,
      dvi,
      dvf,
  )
